# Initial kernel scaffold; baseline (speedup 1.0000x reference)
import jax
import jax.numpy as jnp
from jax.experimental import pallas as pl


def kernel(x):
    raise NotImplementedError("write your pallas kernel here")


# speedup vs baseline: 1.4596x; 1.4596x over previous
import functools

import jax
import jax.numpy as jnp
from jax import lax
from jax.experimental import pallas as pl
from jax.experimental.pallas import tpu as pltpu
from jax.experimental.pallas import tpu_sc as plsc

_D = 512
_SEGS = 512
_SEG_LEN = 32
_TOTAL = _SEGS * _SEG_LEN
_LANES = 16


def _make_sc_kernel():
    info = plsc.get_sparse_core_info()
    nc, ns = info.num_cores, info.num_subcores
    nw = nc * ns
    rows_per_w = _D // nw
    mesh = plsc.VectorSubcoreMesh(core_axis_name="c", subcore_axis_name="s")

    @functools.partial(
        pl.kernel,
        out_type=jax.ShapeDtypeStruct((_D, _SEGS), jnp.float32),
        mesh=mesh,
        scratch_types=[
            pltpu.VMEM((_TOTAL,), jnp.float32),
            pltpu.VMEM((_SEGS,), jnp.float32),
        ],
        compiler_params=pltpu.CompilerParams(needs_layout_passes=False),
    )
    def prod_at(x_hbm, out_hbm, row_v, out_v):
        wid = lax.axis_index("s") * nc + lax.axis_index("c")
        stride_iota = lax.broadcasted_iota(jnp.int32, (_LANES,), 0) * _SEG_LEN

        def row_body(k, carry):
            r = wid * rows_per_w + k
            pltpu.sync_copy(x_hbm.at[r], row_v)

            def grp_body(g, c):
                base = g * (_LANES * _SEG_LEN)
                acc = plsc.load_gather(row_v, [stride_iota + base])
                for i in range(1, _SEG_LEN):
                    acc = acc * plsc.load_gather(row_v, [stride_iota + (base + i)])
                out_v[pl.ds(g * _LANES, _LANES)] = acc
                return c

            lax.fori_loop(0, _SEGS // _LANES, grp_body, 0)
            pltpu.sync_copy(out_v, out_hbm.at[r])
            return carry

        lax.fori_loop(0, rows_per_w, row_body, 0)

    return prod_at


_sc_kernel = _make_sc_kernel()


def kernel(x):
    return _sc_kernel(x)
